# Initial kernel scaffold; baseline (speedup 1.0000x reference)
import jax
import jax.numpy as jnp
from jax.experimental import pallas as pl


def kernel(uid, gid, job, sex, age, pubtime, recent_films, category, fm1_uid, fm1_gid, fm1_job, fm1_sex, fm1_age, fm1_pub, user_emb, group_emb, cat_emb, fm2_job, fm2_sex, fm2_age, fm2_pub, W1, b1, g1, be1, W2, b2, g2, be2, W3, b3, g3, be3, Wo, bo):
    raise NotImplementedError("write your pallas kernel here")


# speedup vs baseline: 7.6866x; 7.6866x over previous
import functools

import jax
import jax.numpy as jnp
from jax import lax
from jax.experimental import pallas as pl
from jax.experimental.pallas import tpu as pltpu
from jax.experimental.pallas import tpu_sc as plsc

B = 4096
HIST = 20
NCAT = 4
EMB = 4

NC, NS, L = 2, 16, 16
NW = NC * NS
BPW = B // NW
NCHUNK = BPW // L

def _align8(n):
    return (n + 7) // 8 * 8

_OFF = {}
_cur = 0
for _name, _len in [
    ("fm1_uid", 944), ("fm1_gid", 1683), ("fm1_job", 22), ("fm1_sex", 2),
    ("fm1_age", 5), ("fm1_pub", 5),
    ("user", 944 * EMB), ("group", 1683 * EMB), ("cat", 20 * EMB),
    ("f2job", 22 * EMB), ("f2sex", 2 * EMB), ("f2age", 5 * EMB),
    ("f2pub", 5 * EMB),
]:
    _OFF[_name] = (_cur, _len)
    _cur = _align8(_cur + _len)
TAB_LEN = _align8(_cur + 8)

_TABLE_ARG_ORDER = ["fm1_uid", "fm1_gid", "fm1_job", "fm1_sex", "fm1_age",
                    "fm1_pub", "user", "group", "cat", "f2job", "f2sex",
                    "f2age", "f2pub"]


def _sc_body(uid_h, gid_h, job_h, sex_h, age_h, pub_h, recent_h, cat_h,
             *rest):
    tabs_h = rest[:13]
    x_out, fm1_out, fm2_out = rest[13:16]
    (tab_v, uid_v, gid_v, job_v, sex_v, age_v, pub_v, recent_v, cat_v,
     x_v, fm1_v, fm2_v) = rest[16:]

    wid = lax.axis_index("s") * NC + lax.axis_index("c")
    base = wid * BPW

    for name, src in zip(_TABLE_ARG_ORDER, tabs_h):
        off, ln = _OFF[name]
        pltpu.sync_copy(src, tab_v.at[pl.ds(off, ln)])
    for src, dst in ((uid_h, uid_v), (gid_h, gid_v), (job_h, job_v),
                     (sex_h, sex_v), (age_h, age_v), (pub_h, pub_v)):
        pltpu.sync_copy(src.at[pl.ds(base, BPW)], dst)
    pltpu.sync_copy(recent_h.at[pl.ds(base, BPW), :], recent_v)
    pltpu.sync_copy(cat_h.at[pl.ds(base, BPW), :], cat_v)

    iota = jnp.arange(L, dtype=jnp.int32)

    def lg(idx):
        return plsc.load_gather(tab_v, [idx])

    def chunk(i, carry):
        sidx = i * L + iota
        uid = plsc.load_gather(uid_v, [sidx])
        gid = plsc.load_gather(gid_v, [sidx])
        job = plsc.load_gather(job_v, [sidx])
        sex = plsc.load_gather(sex_v, [sidx])
        age = plsc.load_gather(age_v, [sidx])
        pub = plsc.load_gather(pub_v, [sidx])

        fm1 = (lg(_OFF["fm1_uid"][0] + uid) + lg(_OFF["fm1_gid"][0] + gid)
               + lg(_OFF["fm1_job"][0] + job) + lg(_OFF["fm1_sex"][0] + sex)
               + lg(_OFF["fm1_age"][0] + age) + lg(_OFF["fm1_pub"][0] + pub))
        plsc.store_scatter(fm1_v, [sidx], fm1)

        f_base = [
            _OFF["f2job"][0] + job * EMB,
            _OFF["f2sex"][0] + sex * EMB,
            _OFF["f2age"][0] + age * EMB,
            _OFF["f2pub"][0] + pub * EMB,
            _OFF["user"][0] + uid * EMB,
            _OFF["group"][0] + gid * EMB,
        ]
        rec_base = [
            _OFF["group"][0]
            + plsc.load_gather(recent_v, [sidx, jnp.full((L,), h, jnp.int32)])
            * EMB
            for h in range(HIST)
        ]
        cat_base = [
            _OFF["cat"][0]
            + plsc.load_gather(cat_v, [sidx, jnp.full((L,), c, jnp.int32)])
            * EMB
            for c in range(NCAT)
        ]

        x_base = sidx * (8 * EMB)
        fm2_acc = jnp.zeros((L,), jnp.float32)
        for d in range(EMB):
            e = [lg(fb + d) for fb in f_base]
            hsum = lg(rec_base[0] + d)
            for rb in rec_base[1:]:
                hsum = hsum + lg(rb + d)
            csum = lg(cat_base[0] + d)
            for cb in cat_base[1:]:
                csum = csum + lg(cb + d)
            e.append(hsum)
            e.append(csum * 0.25)
            s = e[0]
            sq = e[0] * e[0]
            for ef in e[1:]:
                s = s + ef
                sq = sq + ef * ef
            fm2_acc = fm2_acc + (s * s - sq)
            for f in range(8):
                plsc.store_scatter(x_v, [x_base + (f * EMB + d)], e[f])
        plsc.store_scatter(fm2_v, [sidx], fm2_acc * 0.5)
        return carry

    lax.fori_loop(0, NCHUNK, chunk, 0)

    pltpu.sync_copy(x_v, x_out.at[pl.ds(base * (8 * EMB), BPW * 8 * EMB)])
    pltpu.sync_copy(fm1_v, fm1_out.at[pl.ds(base, BPW)])
    pltpu.sync_copy(fm2_v, fm2_out.at[pl.ds(base, BPW)])


_sc_gather = functools.partial(
    pl.kernel,
    out_type=(jax.ShapeDtypeStruct((B * 8 * EMB,), jnp.float32),
              jax.ShapeDtypeStruct((B,), jnp.float32),
              jax.ShapeDtypeStruct((B,), jnp.float32)),
    mesh=plsc.VectorSubcoreMesh(core_axis_name="c", subcore_axis_name="s"),
    scratch_types=[
        pltpu.VMEM((TAB_LEN,), jnp.float32),
        pltpu.VMEM((BPW,), jnp.int32),
        pltpu.VMEM((BPW,), jnp.int32),
        pltpu.VMEM((BPW,), jnp.int32),
        pltpu.VMEM((BPW,), jnp.int32),
        pltpu.VMEM((BPW,), jnp.int32),
        pltpu.VMEM((BPW,), jnp.int32),
        pltpu.VMEM((BPW, HIST), jnp.int32),
        pltpu.VMEM((BPW, NCAT), jnp.int32),
        pltpu.VMEM((BPW * 8 * EMB,), jnp.float32),
        pltpu.VMEM((BPW,), jnp.float32),
        pltpu.VMEM((BPW,), jnp.float32),
    ],
    compiler_params=pltpu.CompilerParams(needs_layout_passes=False),
)(_sc_body)


def _tc_body(x_ref, fm1_ref, fm2_ref, w1_ref, b1_ref, g1_ref, be1_ref,
             w2_ref, b2_ref, g2_ref, be2_ref, w3_ref, b3_ref, g3_ref,
             be3_ref, wo_ref, bo_ref, out_ref):
    def layer(h, w, b, g, be):
        h = jnp.dot(h, w, preferred_element_type=jnp.float32) + b
        m = jnp.mean(h, axis=0, keepdims=True)
        v = jnp.mean((h - m) * (h - m), axis=0, keepdims=True)
        return jnp.maximum(g * (h - m) / jnp.sqrt(v + 1e-5) + be, 0.0)

    h = layer(x_ref[...], w1_ref[...], b1_ref[...], g1_ref[...], be1_ref[...])
    h = layer(h, w2_ref[...], b2_ref[...], g2_ref[...], be2_ref[...])
    h = layer(h, w3_ref[...], b3_ref[...], g3_ref[...], be3_ref[...])
    wo = wo_ref[...]
    out_ref[...] = (jnp.dot(h, wo[:32, :], preferred_element_type=jnp.float32)
                    + fm1_ref[...] * wo[32:33, :]
                    + fm2_ref[...] * wo[33:34, :]
                    + bo_ref[...])


_tc_dnn = pl.pallas_call(
    _tc_body,
    out_shape=jax.ShapeDtypeStruct((B, 5), jnp.float32),
)


def kernel(uid, gid, job, sex, age, pubtime, recent_films, category,
           fm1_uid, fm1_gid, fm1_job, fm1_sex, fm1_age, fm1_pub,
           user_emb, group_emb, cat_emb,
           fm2_job, fm2_sex, fm2_age, fm2_pub,
           W1, b1, g1, be1, W2, b2, g2, be2, W3, b3, g3, be3, Wo, bo):
    tables = [fm1_uid, fm1_gid, fm1_job, fm1_sex, fm1_age, fm1_pub,
              user_emb, group_emb, cat_emb,
              fm2_job, fm2_sex, fm2_age, fm2_pub]
    tables = [t.reshape(-1) for t in tables]
    x_flat, fm1, fm2 = _sc_gather(
        uid.astype(jnp.int32), gid.astype(jnp.int32), job.astype(jnp.int32),
        sex.astype(jnp.int32), age.astype(jnp.int32),
        pubtime.astype(jnp.int32), recent_films.astype(jnp.int32),
        category.astype(jnp.int32), *tables)
    x = x_flat.reshape(B, 8 * EMB)
    return _tc_dnn(x, fm1.reshape(B, 1), fm2.reshape(B, 1),
                   W1, b1.reshape(1, -1), g1.reshape(1, -1),
                   be1.reshape(1, -1),
                   W2, b2.reshape(1, -1), g2.reshape(1, -1),
                   be2.reshape(1, -1),
                   W3, b3.reshape(1, -1), g3.reshape(1, -1),
                   be3.reshape(1, -1),
                   Wo, bo.reshape(1, -1))


# speedup vs baseline: 9.6655x; 1.2575x over previous
import functools

import jax
import jax.numpy as jnp
from jax import lax
from jax.experimental import pallas as pl
from jax.experimental.pallas import tpu as pltpu
from jax.experimental.pallas import tpu_sc as plsc

B = 4096
HIST = 20
NCAT = 4
EMB = 4

NC, NS, L = 2, 16, 16
NW = NC * NS
BPW = B // NW
NCHUNK = BPW // L

def _align8(n):
    return (n + 7) // 8 * 8

_OFF = {}
_cur = 0
for _name, _len in [
    ("fm1_uid", 944), ("fm1_gid", 1683), ("fm1_job", 22), ("fm1_sex", 2),
    ("fm1_age", 5), ("fm1_pub", 5),
    ("user", 944 * EMB), ("group", 1683 * EMB), ("cat", 20 * EMB),
    ("f2job", 22 * EMB), ("f2sex", 2 * EMB), ("f2age", 5 * EMB),
    ("f2pub", 5 * EMB),
]:
    _OFF[_name] = (_cur, _len)
    _cur = _align8(_cur + _len)
TAB_LEN = _align8(_cur + 8)

_TABLE_ARG_ORDER = ["fm1_uid", "fm1_gid", "fm1_job", "fm1_sex", "fm1_age",
                    "fm1_pub", "user", "group", "cat", "f2job", "f2sex",
                    "f2age", "f2pub"]


def _sc_body(idx_h, tab_h, x_out, fm1_out, fm2_out,
             tab_v, idx_v, x_v, fm1_v, fm2_v, sem):
    wid = lax.axis_index("s") * NC + lax.axis_index("c")
    base = wid * BPW

    cps = [pltpu.make_async_copy(tab_h, tab_v, sem),
           pltpu.make_async_copy(idx_h.at[pl.ds(base, BPW), :], idx_v, sem)]
    for c in cps:
        c.start()
    for c in cps:
        c.wait()

    iota = jnp.arange(L, dtype=jnp.int32)

    def lg(idx):
        return plsc.load_gather(tab_v, [idx])

    def col(sidx, c):
        return plsc.load_gather(idx_v, [sidx, jnp.full((L,), c, jnp.int32)])

    def chunk(i, carry):
        sidx = i * L + iota
        uid = col(sidx, 0)
        gid = col(sidx, 1)
        job = col(sidx, 2)
        sex = col(sidx, 3)
        age = col(sidx, 4)
        pub = col(sidx, 5)

        fm1 = (lg(_OFF["fm1_uid"][0] + uid) + lg(_OFF["fm1_gid"][0] + gid)
               + lg(_OFF["fm1_job"][0] + job) + lg(_OFF["fm1_sex"][0] + sex)
               + lg(_OFF["fm1_age"][0] + age) + lg(_OFF["fm1_pub"][0] + pub))
        plsc.store_scatter(fm1_v, [sidx], fm1)

        f_base = [
            _OFF["f2job"][0] + job * EMB,
            _OFF["f2sex"][0] + sex * EMB,
            _OFF["f2age"][0] + age * EMB,
            _OFF["f2pub"][0] + pub * EMB,
            _OFF["user"][0] + uid * EMB,
            _OFF["group"][0] + gid * EMB,
        ]
        rec_base = [_OFF["group"][0] + col(sidx, 6 + h) * EMB
                    for h in range(HIST)]
        cat_base = [_OFF["cat"][0] + col(sidx, 26 + c) * EMB
                    for c in range(NCAT)]

        x_base = sidx * (8 * EMB)
        fm2_acc = jnp.zeros((L,), jnp.float32)
        for d in range(EMB):
            e = [lg(fb + d) for fb in f_base]
            hsum = lg(rec_base[0] + d)
            for rb in rec_base[1:]:
                hsum = hsum + lg(rb + d)
            csum = lg(cat_base[0] + d)
            for cb in cat_base[1:]:
                csum = csum + lg(cb + d)
            e.append(hsum)
            e.append(csum * 0.25)
            s = e[0]
            sq = e[0] * e[0]
            for ef in e[1:]:
                s = s + ef
                sq = sq + ef * ef
            fm2_acc = fm2_acc + (s * s - sq)
            for f in range(8):
                plsc.store_scatter(x_v, [x_base + (f * EMB + d)], e[f])
        plsc.store_scatter(fm2_v, [sidx], fm2_acc * 0.5)
        return carry

    lax.fori_loop(0, NCHUNK, chunk, 0)

    pltpu.sync_copy(x_v, x_out.at[pl.ds(base * (8 * EMB), BPW * 8 * EMB)])
    pltpu.sync_copy(fm1_v, fm1_out.at[pl.ds(base, BPW)])
    pltpu.sync_copy(fm2_v, fm2_out.at[pl.ds(base, BPW)])


_sc_gather = functools.partial(
    pl.kernel,
    out_type=(jax.ShapeDtypeStruct((B * 8 * EMB,), jnp.float32),
              jax.ShapeDtypeStruct((B,), jnp.float32),
              jax.ShapeDtypeStruct((B,), jnp.float32)),
    mesh=plsc.VectorSubcoreMesh(core_axis_name="c", subcore_axis_name="s"),
    scratch_types=[
        pltpu.VMEM((TAB_LEN,), jnp.float32),
        pltpu.VMEM((BPW, 6 + HIST + NCAT), jnp.int32),
        pltpu.VMEM((BPW * 8 * EMB,), jnp.float32),
        pltpu.VMEM((BPW,), jnp.float32),
        pltpu.VMEM((BPW,), jnp.float32),
        pltpu.SemaphoreType.DMA,
    ],
    compiler_params=pltpu.CompilerParams(needs_layout_passes=False),
)(_sc_body)


def _tc_body(x_ref, fm1_ref, fm2_ref, w1_ref, b1_ref, g1_ref, be1_ref,
             w2_ref, b2_ref, g2_ref, be2_ref, w3_ref, b3_ref, g3_ref,
             be3_ref, wo_ref, bo_ref, out_ref):
    def layer(h, w, b, g, be):
        h = jnp.dot(h, w, preferred_element_type=jnp.float32) + b
        m = jnp.mean(h, axis=0, keepdims=True)
        v = jnp.mean((h - m) * (h - m), axis=0, keepdims=True)
        return jnp.maximum(g * (h - m) / jnp.sqrt(v + 1e-5) + be, 0.0)

    h = layer(x_ref[...], w1_ref[...], b1_ref[...], g1_ref[...], be1_ref[...])
    h = layer(h, w2_ref[...], b2_ref[...], g2_ref[...], be2_ref[...])
    h = layer(h, w3_ref[...], b3_ref[...], g3_ref[...], be3_ref[...])
    wo = wo_ref[...]
    out_ref[...] = (jnp.dot(h, wo[:32, :], preferred_element_type=jnp.float32)
                    + fm1_ref[...] * wo[32:33, :]
                    + fm2_ref[...] * wo[33:34, :]
                    + bo_ref[...])


_tc_dnn = pl.pallas_call(
    _tc_body,
    out_shape=jax.ShapeDtypeStruct((B, 5), jnp.float32),
)


def kernel(uid, gid, job, sex, age, pubtime, recent_films, category,
           fm1_uid, fm1_gid, fm1_job, fm1_sex, fm1_age, fm1_pub,
           user_emb, group_emb, cat_emb,
           fm2_job, fm2_sex, fm2_age, fm2_pub,
           W1, b1, g1, be1, W2, b2, g2, be2, W3, b3, g3, be3, Wo, bo):
    tables = [fm1_uid, fm1_gid, fm1_job, fm1_sex, fm1_age, fm1_pub,
              user_emb, group_emb, cat_emb,
              fm2_job, fm2_sex, fm2_age, fm2_pub]
    pieces, cur = [], 0
    for name, t in zip(_TABLE_ARG_ORDER, tables):
        off, ln = _OFF[name]
        if off > cur:
            pieces.append(jnp.zeros((off - cur,), jnp.float32))
        pieces.append(t.reshape(-1))
        cur = off + ln
    if TAB_LEN > cur:
        pieces.append(jnp.zeros((TAB_LEN - cur,), jnp.float32))
    tab = jnp.concatenate(pieces)
    idx_all = jnp.concatenate(
        [uid[:, None], gid[:, None], job[:, None], sex[:, None],
         age[:, None], pubtime[:, None], recent_films, category],
        axis=1).astype(jnp.int32)
    x_flat, fm1, fm2 = _sc_gather(idx_all, tab)
    x = x_flat.reshape(B, 8 * EMB)
    return _tc_dnn(x, fm1.reshape(B, 1), fm2.reshape(B, 1),
                   W1, b1.reshape(1, -1), g1.reshape(1, -1),
                   be1.reshape(1, -1),
                   W2, b2.reshape(1, -1), g2.reshape(1, -1),
                   be2.reshape(1, -1),
                   W3, b3.reshape(1, -1), g3.reshape(1, -1),
                   be3.reshape(1, -1),
                   Wo, bo.reshape(1, -1))


# speedup vs baseline: 10.7637x; 1.1136x over previous
import functools

import jax
import jax.numpy as jnp
from jax import lax
from jax.experimental import pallas as pl
from jax.experimental.pallas import tpu as pltpu
from jax.experimental.pallas import tpu_sc as plsc

B = 4096
HIST = 20
NCAT = 4
EMB = 4
XW = 128

NC, NS, L = 2, 16, 16
NW = NC * NS
BPW = B // NW
NCHUNK = BPW // L

def _align8(n):
    return (n + 7) // 8 * 8

_OFF = {}
_cur = 0
for _name, _len in [
    ("fm1_uid", 944), ("fm1_gid", 1683), ("fm1_job", 22), ("fm1_sex", 2),
    ("fm1_age", 5), ("fm1_pub", 5),
    ("user", 944 * EMB), ("group", 1683 * EMB), ("cat", 20 * EMB),
    ("f2job", 22 * EMB), ("f2sex", 2 * EMB), ("f2age", 5 * EMB),
    ("f2pub", 5 * EMB),
]:
    _OFF[_name] = (_cur, _len)
    _cur = _align8(_cur + _len)
TAB_LEN = _align8(_cur + 8)

_TABLE_ARG_ORDER = ["fm1_uid", "fm1_gid", "fm1_job", "fm1_sex", "fm1_age",
                    "fm1_pub", "user", "group", "cat", "f2job", "f2sex",
                    "f2age", "f2pub"]


def _sc_body(uid_h, gid_h, job_h, sex_h, age_h, pub_h, rec_h, cat_h, tab_h,
             xx_out,
             tab_v, uid_v, gid_v, job_v, sex_v, age_v, pub_v, rec_v, cat_v,
             x_v, sem):
    wid = lax.axis_index("s") * NC + lax.axis_index("c")
    base = wid * BPW

    cps = [pltpu.make_async_copy(tab_h, tab_v, sem),
           pltpu.make_async_copy(uid_h.at[pl.ds(base, BPW)], uid_v, sem),
           pltpu.make_async_copy(gid_h.at[pl.ds(base, BPW)], gid_v, sem),
           pltpu.make_async_copy(job_h.at[pl.ds(base, BPW)], job_v, sem),
           pltpu.make_async_copy(sex_h.at[pl.ds(base, BPW)], sex_v, sem),
           pltpu.make_async_copy(age_h.at[pl.ds(base, BPW)], age_v, sem),
           pltpu.make_async_copy(pub_h.at[pl.ds(base, BPW)], pub_v, sem),
           pltpu.make_async_copy(rec_h.at[pl.ds(base * HIST, BPW * HIST)],
                                 rec_v, sem),
           pltpu.make_async_copy(cat_h.at[pl.ds(base * NCAT, BPW * NCAT)],
                                 cat_v, sem)]
    for c in cps:
        c.start()
    for c in cps:
        c.wait()

    iota = jnp.arange(L, dtype=jnp.int32)

    def lg(idx):
        return plsc.load_gather(tab_v, [idx])

    def put(sidx, col, val):
        plsc.store_scatter(x_v, [sidx, jnp.full((L,), col, jnp.int32)], val)

    @plsc.parallel_loop(0, NCHUNK, unroll=2)
    def chunk(i):
        sidx = i * L + iota
        uid = plsc.load_gather(uid_v, [sidx])
        gid = plsc.load_gather(gid_v, [sidx])
        job = plsc.load_gather(job_v, [sidx])
        sex = plsc.load_gather(sex_v, [sidx])
        age = plsc.load_gather(age_v, [sidx])
        pub = plsc.load_gather(pub_v, [sidx])

        fm1 = (lg(_OFF["fm1_uid"][0] + uid) + lg(_OFF["fm1_gid"][0] + gid)
               + lg(_OFF["fm1_job"][0] + job) + lg(_OFF["fm1_sex"][0] + sex)
               + lg(_OFF["fm1_age"][0] + age) + lg(_OFF["fm1_pub"][0] + pub))
        put(sidx, 32, fm1)

        f_base = [
            _OFF["f2job"][0] + job * EMB,
            _OFF["f2sex"][0] + sex * EMB,
            _OFF["f2age"][0] + age * EMB,
            _OFF["f2pub"][0] + pub * EMB,
            _OFF["user"][0] + uid * EMB,
            _OFF["group"][0] + gid * EMB,
        ]
        rec_i = sidx * HIST
        cat_i = sidx * NCAT
        rec_base = [
            _OFF["group"][0] + plsc.load_gather(rec_v, [rec_i + h]) * EMB
            for h in range(HIST)
        ]
        cat_base = [
            _OFF["cat"][0] + plsc.load_gather(cat_v, [cat_i + c]) * EMB
            for c in range(NCAT)
        ]

        fm2_acc = jnp.zeros((L,), jnp.float32)
        for d in range(EMB):
            e = [lg(fb + d) for fb in f_base]
            hsum = lg(rec_base[0] + d)
            for rb in rec_base[1:]:
                hsum = hsum + lg(rb + d)
            csum = lg(cat_base[0] + d)
            for cb in cat_base[1:]:
                csum = csum + lg(cb + d)
            e.append(hsum)
            e.append(csum * 0.25)
            s = e[0]
            sq = e[0] * e[0]
            for ef in e[1:]:
                s = s + ef
                sq = sq + ef * ef
            fm2_acc = fm2_acc + (s * s - sq)
            for f in range(8):
                put(sidx, f * EMB + d, e[f])
        put(sidx, 33, fm2_acc * 0.5)

    pltpu.sync_copy(x_v, xx_out.at[pl.ds(base, BPW), :])


_sc_gather = functools.partial(
    pl.kernel,
    out_type=jax.ShapeDtypeStruct((B, XW), jnp.float32),
    mesh=plsc.VectorSubcoreMesh(core_axis_name="c", subcore_axis_name="s"),
    scratch_types=[
        pltpu.VMEM((TAB_LEN,), jnp.float32),
        pltpu.VMEM((BPW,), jnp.int32),
        pltpu.VMEM((BPW,), jnp.int32),
        pltpu.VMEM((BPW,), jnp.int32),
        pltpu.VMEM((BPW,), jnp.int32),
        pltpu.VMEM((BPW,), jnp.int32),
        pltpu.VMEM((BPW,), jnp.int32),
        pltpu.VMEM((BPW * HIST,), jnp.int32),
        pltpu.VMEM((BPW * NCAT,), jnp.int32),
        pltpu.VMEM((BPW, XW), jnp.float32),
        pltpu.SemaphoreType.DMA,
    ],
    compiler_params=pltpu.CompilerParams(needs_layout_passes=False),
)(_sc_body)


def _tc_body(xx_ref, w1_ref, b1_ref, g1_ref, be1_ref,
             w2_ref, b2_ref, g2_ref, be2_ref, w3_ref, b3_ref, g3_ref,
             be3_ref, wo_ref, bo_ref, out_ref):
    def layer(h, w, b, g, be):
        n = w.shape[1]
        h = jnp.dot(h, w, preferred_element_type=jnp.float32) + b.reshape(1, n)
        m = jnp.mean(h, axis=0, keepdims=True)
        v = jnp.mean((h - m) * (h - m), axis=0, keepdims=True)
        return jnp.maximum(g.reshape(1, n) * (h - m) / jnp.sqrt(v + 1e-5)
                           + be.reshape(1, n), 0.0)

    xx = xx_ref[...]
    lane = lax.broadcasted_iota(jnp.int32, (B, XW), 1)
    xm = jnp.where(lane < 34, xx, 0.0)
    w1p = jnp.concatenate(
        [w1_ref[...], jnp.zeros((XW - 32, 256), jnp.float32)], axis=0)
    h = layer(xm, w1p, b1_ref[...], g1_ref[...], be1_ref[...])
    h = layer(h, w2_ref[...], b2_ref[...], g2_ref[...], be2_ref[...])
    h = layer(h, w3_ref[...], b3_ref[...], g3_ref[...], be3_ref[...])
    wo = wo_ref[...]
    out_ref[...] = (jnp.dot(h, wo[:32, :], preferred_element_type=jnp.float32)
                    + xm[:, 32:33] * wo[32:33, :]
                    + xm[:, 33:34] * wo[33:34, :]
                    + bo_ref[...].reshape(1, 5))


_tc_dnn = pl.pallas_call(
    _tc_body,
    out_shape=jax.ShapeDtypeStruct((B, 5), jnp.float32),
)


def kernel(uid, gid, job, sex, age, pubtime, recent_films, category,
           fm1_uid, fm1_gid, fm1_job, fm1_sex, fm1_age, fm1_pub,
           user_emb, group_emb, cat_emb,
           fm2_job, fm2_sex, fm2_age, fm2_pub,
           W1, b1, g1, be1, W2, b2, g2, be2, W3, b3, g3, be3, Wo, bo):
    tables = [fm1_uid, fm1_gid, fm1_job, fm1_sex, fm1_age, fm1_pub,
              user_emb, group_emb, cat_emb,
              fm2_job, fm2_sex, fm2_age, fm2_pub]
    pieces, cur = [], 0
    for name, t in zip(_TABLE_ARG_ORDER, tables):
        off, ln = _OFF[name]
        if off > cur:
            pieces.append(jnp.zeros((off - cur,), jnp.float32))
        pieces.append(t.reshape(-1))
        cur = off + ln
    if TAB_LEN > cur:
        pieces.append(jnp.zeros((TAB_LEN - cur,), jnp.float32))
    tab = jnp.concatenate(pieces)
    rec_flat = recent_films.astype(jnp.int32).reshape(-1)
    cat_flat = category.astype(jnp.int32).reshape(-1)
    xx = _sc_gather(
        uid.astype(jnp.int32), gid.astype(jnp.int32), job.astype(jnp.int32),
        sex.astype(jnp.int32), age.astype(jnp.int32),
        pubtime.astype(jnp.int32), rec_flat, cat_flat, tab)
    return _tc_dnn(xx, W1, b1, g1, be1, W2, b2, g2, be2,
                   W3, b3, g3, be3, Wo, bo)


# speedup vs baseline: 11.0310x; 1.0248x over previous
import functools

import jax
import jax.numpy as jnp
from jax import lax
from jax.experimental import pallas as pl
from jax.experimental.pallas import tpu as pltpu
from jax.experimental.pallas import tpu_sc as plsc

B = 4096
HIST = 20
NCAT = 4
EMB = 4
XW = 128
MW = 32

NC, NS, L = 2, 16, 16
NW = NC * NS
BPW = B // NW
NCHUNK = BPW // L

_REC0 = 0
_CAT0 = HIST
_UID, _GID, _JOB, _SEX, _AGE, _PUB = 24, 25, 26, 27, 28, 29

_OFF = {}
_cur = 0
for _name, _len in [
    ("fm1_uid", 944), ("fm1_gid", 1683), ("fm1_job", 22), ("fm1_sex", 2),
    ("fm1_age", 5), ("fm1_pub", 5),
    ("user", 944 * EMB), ("group", 1683 * EMB), ("cat", 20 * EMB),
    ("f2job", 22 * EMB), ("f2sex", 2 * EMB), ("f2age", 5 * EMB),
    ("f2pub", 5 * EMB),
]:
    _OFF[_name] = _cur
    _cur += _len
TAB_LEN = (_cur + 7) // 8 * 8

_TABLE_ARG_ORDER = ["fm1_uid", "fm1_gid", "fm1_job", "fm1_sex", "fm1_age",
                    "fm1_pub", "user", "group", "cat", "f2job", "f2sex",
                    "f2age", "f2pub"]


def _sc_body(meta_h, tab_h, xx_out, meta_v, tab_v, x_v, sem):
    wid = lax.axis_index("s") * NC + lax.axis_index("c")
    base = wid * BPW

    mrow = pl.multiple_of(base * MW // 128, 8)
    cps = [pltpu.make_async_copy(meta_h.at[pl.ds(mrow, BPW * MW // 128), :],
                                 meta_v, sem),
           pltpu.make_async_copy(tab_h, tab_v, sem)]
    for c in cps:
        c.start()
    for c in cps:
        c.wait()

    iota = jnp.arange(L, dtype=jnp.int32)

    def put(sidx, col, val):
        plsc.store_scatter(x_v, [sidx, jnp.full((L,), col, jnp.int32)], val)

    @plsc.parallel_loop(0, NCHUNK, unroll=4)
    def chunk(i):
        sidx = i * L + iota
        mr = lax.shift_right_logical(sidx, 2)
        mc = lax.shift_left(jnp.bitwise_and(sidx, 3), 5)

        def midx(c):
            return plsc.load_gather(meta_v, [mr, mc + c])

        uid = midx(_UID)
        gid = midx(_GID)
        job = midx(_JOB)
        sex = midx(_SEX)
        age = midx(_AGE)
        pub = midx(_PUB)

        def lg(idx):
            return plsc.load_gather(tab_v, [idx])

        fm1 = (lg(_OFF["fm1_uid"] + uid) + lg(_OFF["fm1_gid"] + gid)
               + lg(_OFF["fm1_job"] + job) + lg(_OFF["fm1_sex"] + sex)
               + lg(_OFF["fm1_age"] + age) + lg(_OFF["fm1_pub"] + pub))
        put(sidx, 32, fm1)

        f_base = [
            _OFF["f2job"] + job * EMB,
            _OFF["f2sex"] + sex * EMB,
            _OFF["f2age"] + age * EMB,
            _OFF["f2pub"] + pub * EMB,
            _OFF["user"] + uid * EMB,
            _OFF["group"] + gid * EMB,
        ]
        rec_base = [_OFF["group"] + midx(_REC0 + h) * EMB
                    for h in range(HIST)]
        cat_base = [_OFF["cat"] + midx(_CAT0 + c) * EMB
                    for c in range(NCAT)]

        fm2_acc = jnp.zeros((L,), jnp.float32)
        for d in range(EMB):
            e = [lg(fb + d) for fb in f_base]
            hsum = lg(rec_base[0] + d)
            for rb in rec_base[1:]:
                hsum = hsum + lg(rb + d)
            csum = lg(cat_base[0] + d)
            for cb in cat_base[1:]:
                csum = csum + lg(cb + d)
            e.append(hsum)
            e.append(csum * 0.25)
            s = e[0]
            sq = e[0] * e[0]
            for ef in e[1:]:
                s = s + ef
                sq = sq + ef * ef
            fm2_acc = fm2_acc + (s * s - sq)
            for f in range(8):
                put(sidx, f * EMB + d, e[f])
        put(sidx, 33, fm2_acc * 0.5)

    pltpu.sync_copy(x_v, xx_out.at[pl.ds(base, BPW), :])


_sc_gather = functools.partial(
    pl.kernel,
    out_type=jax.ShapeDtypeStruct((B, XW), jnp.float32),
    mesh=plsc.VectorSubcoreMesh(core_axis_name="c", subcore_axis_name="s"),
    scratch_types=[
        pltpu.VMEM((BPW * MW // 128, 128), jnp.int32),
        pltpu.VMEM((TAB_LEN,), jnp.float32),
        pltpu.VMEM((BPW, XW), jnp.float32),
        pltpu.SemaphoreType.DMA,
    ],
    compiler_params=pltpu.CompilerParams(needs_layout_passes=False),
)(_sc_body)


def _tc_body(xx_ref, w1_ref, b1_ref, g1_ref, be1_ref,
             w2_ref, b2_ref, g2_ref, be2_ref, w3_ref, b3_ref, g3_ref,
             be3_ref, wo_ref, bo_ref, out_ref):
    def layer(h, w, b, g, be):
        n = w.shape[1]
        h = jnp.dot(h, w, preferred_element_type=jnp.float32) + b.reshape(1, n)
        m = jnp.mean(h, axis=0, keepdims=True)
        v = jnp.mean((h - m) * (h - m), axis=0, keepdims=True)
        return jnp.maximum(g.reshape(1, n) * (h - m) / jnp.sqrt(v + 1e-5)
                           + be.reshape(1, n), 0.0)

    xx = xx_ref[...]
    lane = lax.broadcasted_iota(jnp.int32, (B, XW), 1)
    xm = jnp.where(lane < 34, xx, 0.0)
    w1p = jnp.concatenate(
        [w1_ref[...], jnp.zeros((XW - 32, 256), jnp.float32)], axis=0)
    h = layer(xm, w1p, b1_ref[...], g1_ref[...], be1_ref[...])
    h = layer(h, w2_ref[...], b2_ref[...], g2_ref[...], be2_ref[...])
    h = layer(h, w3_ref[...], b3_ref[...], g3_ref[...], be3_ref[...])
    wo = wo_ref[...]
    out_ref[...] = (jnp.dot(h, wo[:32, :], preferred_element_type=jnp.float32)
                    + xm[:, 32:33] * wo[32:33, :]
                    + xm[:, 33:34] * wo[33:34, :]
                    + bo_ref[...].reshape(1, 5))


_tc_dnn = pl.pallas_call(
    _tc_body,
    out_shape=jax.ShapeDtypeStruct((B, 5), jnp.float32),
)


def kernel(uid, gid, job, sex, age, pubtime, recent_films, category,
           fm1_uid, fm1_gid, fm1_job, fm1_sex, fm1_age, fm1_pub,
           user_emb, group_emb, cat_emb,
           fm2_job, fm2_sex, fm2_age, fm2_pub,
           W1, b1, g1, be1, W2, b2, g2, be2, W3, b3, g3, be3, Wo, bo):
    i32 = jnp.int32
    meta = jnp.concatenate(
        [recent_films.astype(i32), category.astype(i32),
         uid.astype(i32)[:, None], gid.astype(i32)[:, None],
         job.astype(i32)[:, None], sex.astype(i32)[:, None],
         age.astype(i32)[:, None], pubtime.astype(i32)[:, None],
         jnp.zeros((B, 2), i32)], axis=1).reshape(B * MW // 128, 128)
    tables = dict(fm1_uid=fm1_uid, fm1_gid=fm1_gid, fm1_job=fm1_job,
                  fm1_sex=fm1_sex, fm1_age=fm1_age, fm1_pub=fm1_pub,
                  user=user_emb, group=group_emb, cat=cat_emb,
                  f2job=fm2_job, f2sex=fm2_sex, f2age=fm2_age,
                  f2pub=fm2_pub)
    pieces = [tables[name].reshape(-1) for name in _TABLE_ARG_ORDER]
    pieces.append(jnp.zeros((TAB_LEN - _cur,), jnp.float32))
    tab = jnp.concatenate(pieces)
    xx = _sc_gather(meta, tab)
    return _tc_dnn(xx, W1, b1, g1, be1, W2, b2, g2, be2,
                   W3, b3, g3, be3, Wo, bo)


# speedup vs baseline: 11.3178x; 1.0260x over previous
import functools

import jax
import jax.numpy as jnp
from jax import lax
from jax.experimental import pallas as pl
from jax.experimental.pallas import tpu as pltpu
from jax.experimental.pallas import tpu_sc as plsc

B = 4096
HIST = 20
NCAT = 4
EMB = 4
XW = 128
MW = 32

NC, NS, L = 2, 16, 16
NW = NC * NS
BPW = B // NW
NCHUNK = BPW // L

_REC0 = 0
_CAT0 = HIST
_UID, _GID, _JOB, _SEX, _AGE, _PUB = 24, 25, 26, 27, 28, 29

_OFF = {}
_cur = 0
for _name, _len in [
    ("fm1_uid", 944), ("fm1_gid", 1683), ("fm1_job", 22), ("fm1_sex", 2),
    ("fm1_age", 5), ("fm1_pub", 5),
    ("user", 944 * EMB), ("group", 1683 * EMB), ("cat", 20 * EMB),
    ("f2job", 22 * EMB), ("f2sex", 2 * EMB), ("f2age", 5 * EMB),
    ("f2pub", 5 * EMB),
]:
    _OFF[_name] = _cur
    _cur += _len
TAB_LEN = (_cur + 7) // 8 * 8

_TABLE_ARG_ORDER = ["fm1_uid", "fm1_gid", "fm1_job", "fm1_sex", "fm1_age",
                    "fm1_pub", "user", "group", "cat", "f2job", "f2sex",
                    "f2age", "f2pub"]


def _sc_body(meta_h, tab_h, xx_out, meta_v, tab_v, x_v, sem):
    wid = lax.axis_index("s") * NC + lax.axis_index("c")
    base = wid * BPW

    mrow = pl.multiple_of(base * MW // 128, 8)
    cps = [pltpu.make_async_copy(meta_h.at[pl.ds(mrow, BPW * MW // 128), :],
                                 meta_v, sem),
           pltpu.make_async_copy(tab_h, tab_v, sem)]
    for c in cps:
        c.start()
    for c in cps:
        c.wait()

    iota = jnp.arange(L, dtype=jnp.int32)

    def put(xb, col, val):
        plsc.store_scatter(x_v, [xb + col], val)

    @plsc.parallel_loop(0, NCHUNK, unroll=2)
    def chunk(i):
        sidx = i * L + iota
        xb = lax.shift_left(sidx, 7)
        mr = lax.shift_right_logical(sidx, 2)
        mc = lax.shift_left(jnp.bitwise_and(sidx, 3), 5)

        def midx(c):
            return plsc.load_gather(meta_v, [mr, mc + c])

        uid = midx(_UID)
        gid = midx(_GID)
        job = midx(_JOB)
        sex = midx(_SEX)
        age = midx(_AGE)
        pub = midx(_PUB)

        def lg(idx):
            return plsc.load_gather(tab_v, [idx])

        fm1 = (lg(_OFF["fm1_uid"] + uid) + lg(_OFF["fm1_gid"] + gid)
               + lg(_OFF["fm1_job"] + job) + lg(_OFF["fm1_sex"] + sex)
               + lg(_OFF["fm1_age"] + age) + lg(_OFF["fm1_pub"] + pub))
        put(xb, 32, fm1)

        f_base = [
            _OFF["f2job"] + job * EMB,
            _OFF["f2sex"] + sex * EMB,
            _OFF["f2age"] + age * EMB,
            _OFF["f2pub"] + pub * EMB,
            _OFF["user"] + uid * EMB,
            _OFF["group"] + gid * EMB,
        ]
        rec_base = [_OFF["group"] + midx(_REC0 + h) * EMB
                    for h in range(HIST)]
        cat_base = [_OFF["cat"] + midx(_CAT0 + c) * EMB
                    for c in range(NCAT)]

        fm2_acc = jnp.zeros((L,), jnp.float32)
        for d in range(EMB):
            e = [lg(fb + d) for fb in f_base]
            hsum = lg(rec_base[0] + d)
            for rb in rec_base[1:]:
                hsum = hsum + lg(rb + d)
            csum = lg(cat_base[0] + d)
            for cb in cat_base[1:]:
                csum = csum + lg(cb + d)
            e.append(hsum)
            e.append(csum * 0.25)
            s = e[0]
            sq = e[0] * e[0]
            for ef in e[1:]:
                s = s + ef
                sq = sq + ef * ef
            fm2_acc = fm2_acc + (s * s - sq)
            for f in range(8):
                put(xb, f * EMB + d, e[f])
        put(xb, 33, fm2_acc * 0.5)

    pltpu.sync_copy(x_v, xx_out.at[pl.ds(base * XW, BPW * XW)])


_sc_gather = functools.partial(
    pl.kernel,
    out_type=jax.ShapeDtypeStruct((B * XW,), jnp.float32),
    mesh=plsc.VectorSubcoreMesh(core_axis_name="c", subcore_axis_name="s"),
    scratch_types=[
        pltpu.VMEM((BPW * MW // 128, 128), jnp.int32),
        pltpu.VMEM((TAB_LEN,), jnp.float32),
        pltpu.VMEM((BPW * XW,), jnp.float32),
        pltpu.SemaphoreType.DMA,
    ],
    compiler_params=pltpu.CompilerParams(needs_layout_passes=False),
)(_sc_body)


def _tc_body(xx_ref, w1_ref, b1_ref, g1_ref, be1_ref,
             w2_ref, b2_ref, g2_ref, be2_ref, w3_ref, b3_ref, g3_ref,
             be3_ref, wo_ref, bo_ref, out_ref):
    def layer(h, w, b, g, be):
        n = w.shape[1]
        h = jnp.dot(h, w, preferred_element_type=jnp.float32) + b.reshape(1, n)
        m = jnp.mean(h, axis=0, keepdims=True)
        v = jnp.mean((h - m) * (h - m), axis=0, keepdims=True)
        return jnp.maximum(g.reshape(1, n) * (h - m) / jnp.sqrt(v + 1e-5)
                           + be.reshape(1, n), 0.0)

    xx = jnp.reshape(xx_ref[...], (B, XW))
    lane = lax.broadcasted_iota(jnp.int32, (B, XW), 1)
    xm = jnp.where(lane < 34, xx, 0.0)
    w1p = jnp.concatenate(
        [w1_ref[...], jnp.zeros((XW - 32, 256), jnp.float32)], axis=0)
    h = layer(xm, w1p, b1_ref[...], g1_ref[...], be1_ref[...])
    h = layer(h, w2_ref[...], b2_ref[...], g2_ref[...], be2_ref[...])
    h = layer(h, w3_ref[...], b3_ref[...], g3_ref[...], be3_ref[...])
    wo = wo_ref[...]
    out_ref[...] = (jnp.dot(h, wo[:32, :], preferred_element_type=jnp.float32)
                    + xm[:, 32:33] * wo[32:33, :]
                    + xm[:, 33:34] * wo[33:34, :]
                    + bo_ref[...].reshape(1, 5))


_tc_dnn = pl.pallas_call(
    _tc_body,
    out_shape=jax.ShapeDtypeStruct((B, 5), jnp.float32),
)


def kernel(uid, gid, job, sex, age, pubtime, recent_films, category,
           fm1_uid, fm1_gid, fm1_job, fm1_sex, fm1_age, fm1_pub,
           user_emb, group_emb, cat_emb,
           fm2_job, fm2_sex, fm2_age, fm2_pub,
           W1, b1, g1, be1, W2, b2, g2, be2, W3, b3, g3, be3, Wo, bo):
    i32 = jnp.int32
    meta = jnp.concatenate(
        [recent_films.astype(i32), category.astype(i32),
         uid.astype(i32)[:, None], gid.astype(i32)[:, None],
         job.astype(i32)[:, None], sex.astype(i32)[:, None],
         age.astype(i32)[:, None], pubtime.astype(i32)[:, None],
         jnp.zeros((B, 2), i32)], axis=1).reshape(B * MW // 128, 128)
    tables = dict(fm1_uid=fm1_uid, fm1_gid=fm1_gid, fm1_job=fm1_job,
                  fm1_sex=fm1_sex, fm1_age=fm1_age, fm1_pub=fm1_pub,
                  user=user_emb, group=group_emb, cat=cat_emb,
                  f2job=fm2_job, f2sex=fm2_sex, f2age=fm2_age,
                  f2pub=fm2_pub)
    pieces = [tables[name].reshape(-1) for name in _TABLE_ARG_ORDER]
    pieces.append(jnp.zeros((TAB_LEN - _cur,), jnp.float32))
    tab = jnp.concatenate(pieces)
    xx = _sc_gather(meta, tab)
    return _tc_dnn(xx, W1, b1, g1, be1, W2, b2, g2, be2,
                   W3, b3, g3, be3, Wo, bo)


# speedup vs baseline: 12.4738x; 1.1021x over previous
import functools

import jax
import jax.numpy as jnp
from jax import lax
from jax.experimental import pallas as pl
from jax.experimental.pallas import tpu as pltpu
from jax.experimental.pallas import tpu_sc as plsc

B = 4096
HIST = 20
NCAT = 4
EMB = 4
XW = 128
MW = 32

NC, NS, L = 2, 16, 16
NW = NC * NS
BPW = B // NW
NCHUNK = BPW // L

_REC0 = 0
_CAT0 = HIST
_UID, _GID, _JOB, _SEX, _AGE, _PUB = 24, 25, 26, 27, 28, 29

_OFF = {}
_cur = 0
for _name, _len in [
    ("fm1_uid", 944), ("fm1_gid", 1683), ("fm1_job", 22), ("fm1_sex", 2),
    ("fm1_age", 5), ("fm1_pub", 5),
    ("user", 944 * EMB), ("group", 1683 * EMB), ("cat", 20 * EMB),
    ("f2job", 22 * EMB), ("f2sex", 2 * EMB), ("f2age", 5 * EMB),
    ("f2pub", 5 * EMB),
]:
    _OFF[_name] = _cur
    _cur += _len
TAB_LEN = (_cur + 7) // 8 * 8

_TABLE_ARG_ORDER = ["fm1_uid", "fm1_gid", "fm1_job", "fm1_sex", "fm1_age",
                    "fm1_pub", "user", "group", "cat", "f2job", "f2sex",
                    "f2age", "f2pub"]


def _sc_body(meta_h, tab_h, xx_out, meta_v, tab_v, x_v, sem):
    wid = lax.axis_index("s") * NC + lax.axis_index("c")
    base = wid * BPW

    mrow = pl.multiple_of(base * MW // 128, 8)
    cps = [pltpu.make_async_copy(meta_h.at[pl.ds(mrow, BPW * MW // 128), :],
                                 meta_v, sem),
           pltpu.make_async_copy(tab_h, tab_v, sem)]
    for c in cps:
        c.start()
    for c in cps:
        c.wait()

    iota = jnp.arange(L, dtype=jnp.int32)

    def put(xb, col, val):
        plsc.store_scatter(x_v, [xb + col], val)

    @plsc.parallel_loop(0, NCHUNK, unroll=2)
    def chunk(i):
        sidx = i * L + iota
        xb = lax.shift_left(sidx, 7)
        mr = lax.shift_right_logical(sidx, 2)
        mc = lax.shift_left(jnp.bitwise_and(sidx, 3), 5)

        def midx(c):
            return plsc.load_gather(meta_v, [mr, mc + c])

        uid = midx(_UID)
        gid = midx(_GID)
        job = midx(_JOB)
        sex = midx(_SEX)
        age = midx(_AGE)
        pub = midx(_PUB)

        def lg(idx):
            return plsc.load_gather(tab_v, [idx])

        fm1 = (lg(_OFF["fm1_uid"] + uid) + lg(_OFF["fm1_gid"] + gid)
               + lg(_OFF["fm1_job"] + job) + lg(_OFF["fm1_sex"] + sex)
               + lg(_OFF["fm1_age"] + age) + lg(_OFF["fm1_pub"] + pub))
        put(xb, 32, fm1)

        f_base = [
            _OFF["f2job"] + job * EMB,
            _OFF["f2sex"] + sex * EMB,
            _OFF["f2age"] + age * EMB,
            _OFF["f2pub"] + pub * EMB,
            _OFF["user"] + uid * EMB,
            _OFF["group"] + gid * EMB,
        ]
        rec_base = [_OFF["group"] + midx(_REC0 + h) * EMB
                    for h in range(HIST)]
        cat_base = [_OFF["cat"] + midx(_CAT0 + c) * EMB
                    for c in range(NCAT)]

        fm2_acc = jnp.zeros((L,), jnp.float32)
        for d in range(EMB):
            e = [lg(fb + d) for fb in f_base]
            hsum = lg(rec_base[0] + d)
            for rb in rec_base[1:]:
                hsum = hsum + lg(rb + d)
            csum = lg(cat_base[0] + d)
            for cb in cat_base[1:]:
                csum = csum + lg(cb + d)
            e.append(hsum)
            e.append(csum * 0.25)
            s = e[0]
            sq = e[0] * e[0]
            for ef in e[1:]:
                s = s + ef
                sq = sq + ef * ef
            fm2_acc = fm2_acc + (s * s - sq)
            for f in range(8):
                put(xb, f * EMB + d, e[f])
        put(xb, 33, fm2_acc * 0.5)

    pltpu.sync_copy(x_v, xx_out.at[pl.ds(base * XW, BPW * XW)])


_sc_gather = functools.partial(
    pl.kernel,
    out_type=jax.ShapeDtypeStruct((B * XW,), jnp.float32),
    mesh=plsc.VectorSubcoreMesh(core_axis_name="c", subcore_axis_name="s"),
    scratch_types=[
        pltpu.VMEM((BPW * MW // 128, 128), jnp.int32),
        pltpu.VMEM((TAB_LEN,), jnp.float32),
        pltpu.VMEM((BPW * XW,), jnp.float32),
        pltpu.SemaphoreType.DMA,
    ],
    compiler_params=pltpu.CompilerParams(needs_layout_passes=False),
)(_sc_body)


def _dotT(a, b):
    return lax.dot_general(a, b, (((1,), (1,)), ((), ())),
                           preferred_element_type=jnp.float32)


def _tc_body(xx_ref, w1_ref, b1_ref, g1_ref, be1_ref,
             w2_ref, b2_ref, g2_ref, be2_ref, w3t_ref, b3_ref, g3_ref,
             be3_ref, wot_ref, bo_ref, out_ref):
    ones = jnp.full((1, B), 1.0 / B, jnp.float32)

    def bn_relu(h, b, g, be):
        n = h.shape[1]
        h = h + b.reshape(1, n)
        m = jnp.dot(ones, h, preferred_element_type=jnp.float32)
        ms = jnp.dot(ones, h * h, preferred_element_type=jnp.float32)
        v = ms - m * m
        scale = g.reshape(1, n) / jnp.sqrt(v + 1e-5)
        shift = be.reshape(1, n) - m * scale
        return jnp.maximum(h * scale + shift, 0.0)

    xx = jnp.reshape(xx_ref[...], (B, XW))
    lane = lax.broadcasted_iota(jnp.int32, (B, XW), 1)
    xm = jnp.where(lane < 34, xx, 0.0)
    w1p = jnp.concatenate(
        [w1_ref[...], jnp.zeros((XW - 32, 256), jnp.float32)], axis=0)
    h = bn_relu(jnp.dot(xm, w1p, preferred_element_type=jnp.float32),
                b1_ref[...], g1_ref[...], be1_ref[...])
    h = bn_relu(jnp.dot(h, w2_ref[...], preferred_element_type=jnp.float32),
                b2_ref[...], g2_ref[...], be2_ref[...])
    h = bn_relu(_dotT(h, w3t_ref[...]),
                b3_ref[...], g3_ref[...], be3_ref[...])
    wot = wot_ref[...]
    wofm = jnp.concatenate(
        [jnp.zeros((5, 32), jnp.float32), wot[:, 32:34],
         jnp.zeros((5, XW - 34), jnp.float32)], axis=1)
    outt = (_dotT(wot[:, :32], h) + _dotT(wofm, xm)
            + lax.broadcast_in_dim(bo_ref[...], (5, B), (0,)))
    out_ref[...] = outt


_tc_dnn = pl.pallas_call(
    _tc_body,
    out_shape=jax.ShapeDtypeStruct((5, B), jnp.float32),
)


def kernel(uid, gid, job, sex, age, pubtime, recent_films, category,
           fm1_uid, fm1_gid, fm1_job, fm1_sex, fm1_age, fm1_pub,
           user_emb, group_emb, cat_emb,
           fm2_job, fm2_sex, fm2_age, fm2_pub,
           W1, b1, g1, be1, W2, b2, g2, be2, W3, b3, g3, be3, Wo, bo):
    i32 = jnp.int32
    meta = jnp.concatenate(
        [recent_films.astype(i32), category.astype(i32),
         uid.astype(i32)[:, None], gid.astype(i32)[:, None],
         job.astype(i32)[:, None], sex.astype(i32)[:, None],
         age.astype(i32)[:, None], pubtime.astype(i32)[:, None],
         jnp.zeros((B, 2), i32)], axis=1).reshape(B * MW // 128, 128)
    tables = dict(fm1_uid=fm1_uid, fm1_gid=fm1_gid, fm1_job=fm1_job,
                  fm1_sex=fm1_sex, fm1_age=fm1_age, fm1_pub=fm1_pub,
                  user=user_emb, group=group_emb, cat=cat_emb,
                  f2job=fm2_job, f2sex=fm2_sex, f2age=fm2_age,
                  f2pub=fm2_pub)
    pieces = [tables[name].reshape(-1) for name in _TABLE_ARG_ORDER]
    pieces.append(jnp.zeros((TAB_LEN - _cur,), jnp.float32))
    tab = jnp.concatenate(pieces)
    xx = _sc_gather(meta, tab)
    outt = _tc_dnn(xx, W1, b1, g1, be1, W2, b2, g2, be2,
                   W3.T, b3, g3, be3, Wo.T, bo)
    return jnp.transpose(outt)
